# baseline (speedup 1.0000x reference)
import functools

import jax
import jax.numpy as jnp
from jax import lax
from jax.experimental import pallas as pl
from jax.experimental.pallas import tpu as pltpu
from jax.experimental.pallas import tpu_sc as plsc

DP = 144
NPAD = 10240
C = 80
R = 3
NC = 2
NS = 16
BH = 400
BN = 1024

E_TOTAL = 320000


def _pre_body(h_ref, p2_ref, wfc_ref, wsd_ref, wq_ref,
              z_ref, sd_ref, q2_ref, smax_ref, dmax_ref, qmax_ref):
    i = pl.program_id(0)
    z = lax.dot_general(h_ref[...], wfc_ref[...], (((1,), (1,)), ((), ())),
                        preferred_element_type=jnp.float32)
    sd = jnp.dot(z, wsd_ref[...], preferred_element_type=jnp.float32)
    q2 = jnp.dot(p2_ref[...], wq_ref[...], preferred_element_type=jnp.float32)
    ones = jnp.ones((z.shape[0], 1), jnp.float32)
    zeros = jnp.zeros((z.shape[0], DP - 130), jnp.float32)
    z_ref[...] = jnp.concatenate([z, ones, sd[:, 0:1], zeros], axis=1)
    sd_ref[...] = sd
    q2_ref[...] = q2
    sm, dm, qm = jnp.max(sd[:, 0]), jnp.max(sd[:, 1]), jnp.max(q2)

    @pl.when(i == 0)
    def _():
        smax_ref[0, 0] = sm
        dmax_ref[0, 0] = dm
        qmax_ref[0, 0] = qm

    @pl.when(i > 0)
    def _():
        smax_ref[0, 0] = jnp.maximum(smax_ref[0, 0], sm)
        dmax_ref[0, 0] = jnp.maximum(dmax_ref[0, 0], dm)
        qmax_ref[0, 0] = jnp.maximum(qmax_ref[0, 0], qm)


def _post_body(u_ref, o_ref):
    a = u_ref[0] + u_ref[1]
    num = a[:, :128]
    den = a[:, 128:129]
    o_ref[...] = jnp.where(den > 0.0, num / den, 0.0)


def _qd_body(d_hbm, ei_hbm, q_hbm, qd_out, d_v, dst_t, qd_t):
    ci = lax.axis_index("c")
    si = lax.axis_index("s")
    wid = ci * NS + si
    ept = E_TOTAL // (NC * NS)
    base = pl.multiple_of(wid * ept, 8)
    pltpu.sync_copy(d_hbm, d_v)
    pltpu.sync_copy(ei_hbm.at[1, pl.ds(base, ept)], dst_t)
    pltpu.sync_copy(q_hbm.at[pl.ds(base, ept)], qd_t)

    def _grp(i, _):
        sl = pl.ds(i * 16, 16)
        dv = plsc.load_gather(d_v, [dst_t[sl]])
        qd_t[sl] = qd_t[sl] + dv
        return 0

    lax.fori_loop(0, ept // 16, _grp, 0)
    pltpu.sync_copy(qd_t, qd_out.at[pl.ds(base, ept)])


def _sc_body(z_hbm, qd_hbm, ei_hbm, cvec_hbm,
             u_out,
             c_v, src_b, dst_b, dst_s, qd_b, w_b, rows_b, u_sh,
             lsem, gsem, ssem):
    ci = lax.axis_index("c")
    si = lax.axis_index("s")
    wid = ci * NS + si
    ept = E_TOTAL // (NC * NS)
    nch = ept // C
    rows_per_tile = NPAD // NS
    tbase = pl.multiple_of(wid * ept, 8)

    def _zrows(i, _):
        for j in range(DP // 16):
            rows_b[0][i, pl.ds(j * 16, 16)] = jnp.zeros((16,), jnp.float32)
        return 0
    lax.fori_loop(0, C, _zrows, 0)

    def _zacc(i, _):
        off = pl.multiple_of(si * rows_per_tile + i * C, 8)
        pltpu.sync_copy(rows_b[0], u_sh.at[pl.ds(off, C), :])
        return 0
    lax.fori_loop(0, rows_per_tile // C, _zacc, 0)

    pltpu.sync_copy(cvec_hbm, c_v)
    plsc.subcore_barrier()

    cvec = c_v[...]
    mask0 = jnp.where(lax.iota(jnp.int32, 16) == 0, 1.0, 0.0)

    def _lin_start(g, b):
        off = pl.ds(tbase + g * C, C)
        pltpu.async_copy(ei_hbm.at[0, off], src_b[b], lsem[b])
        pltpu.async_copy(ei_hbm.at[1, off], dst_b[b], lsem[b])
        pltpu.async_copy(qd_hbm.at[off], qd_b[b], lsem[b])

    def _lin_wait(g, b):
        off = pl.ds(tbase + g * C, C)
        pltpu.make_async_copy(ei_hbm.at[0, off], src_b[b], lsem[b]).wait()
        pltpu.make_async_copy(ei_hbm.at[1, off], dst_b[b], lsem[b]).wait()
        pltpu.make_async_copy(qd_hbm.at[off], qd_b[b], lsem[b]).wait()

    def _gather_start(b):
        pltpu.async_copy(z_hbm.at[src_b[b]], rows_b[b], gsem[b])

    def _gather_wait(b):
        pltpu.make_async_copy(z_hbm.at[src_b[b]], rows_b[b],
                              gsem[b]).wait()

    def _scatter_start(b):
        def _cp(i, _):
            sl = pl.ds(i * 16, 16)
            dst_s[b][sl] = dst_b[b][sl]
            return 0
        lax.fori_loop(0, C // 16, _cp, 0)
        pltpu.async_copy(rows_b[b], u_sh.at[dst_s[b]], ssem[b], add=True)

    def _scatter_wait(b):
        pltpu.make_async_copy(rows_b[b], u_sh.at[dst_s[b]],
                              ssem[b]).wait()

    def _compute_chunk(b):
        def _wgrp(g16, _):
            sl = pl.ds(g16 * 16, 16)
            eids = lax.iota(jnp.int32, 16) + g16 * 16
            col = jnp.full((16,), 129, jnp.int32)
            sv = plsc.load_gather(rows_b[b], [eids, col])
            a = sv + qd_b[b][sl]
            e = jnp.where(a >= 0.0, a, a * 0.01)
            w_b[b][sl] = jnp.exp(e - cvec)
            return 0
        lax.fori_loop(0, C // 16, _wgrp, 0)

        def _sgrp(g16, _):
            wv = w_b[b][pl.ds(g16 * 16, 16)]
            ebase = g16 * 16
            for lane in range(16):
                bw = jnp.full((16,), wv[lane], jnp.float32)
                ei = ebase + lane
                for j in range(128 // 16):
                    sl2 = pl.ds(j * 16, 16)
                    rows_b[b][ei, sl2] = rows_b[b][ei, sl2] * bw
                rows_b[b][ei, pl.ds(128, 16)] = bw * mask0
            return 0
        lax.fori_loop(0, C // 16, _sgrp, 0)

    for b in range(R):
        _lin_start(b, b)
    _lin_wait(0, 0)
    _gather_start(0)
    _lin_wait(1, 1)
    _gather_start(1)

    def _iter(i, _):
        for u in range(R):
            g = i * R + u
            b = u

            @pl.when(g < nch)
            def _():
                _gather_wait(b)
                _compute_chunk(b)
                _scatter_start(b)

                @pl.when(g + R < nch)
                def _():
                    _lin_start(g + R, b)

                nb = (u + 2) % R

                @pl.when((g + 2 < nch) & (g >= 1))
                def _():
                    _scatter_wait(nb)
                    _lin_wait(g + 2, nb)
                    _gather_start(nb)

                @pl.when((g + 2 < nch) & (g < 1))
                def _():
                    _lin_wait(g + 2, nb)
                    _gather_start(nb)
        return 0

    lax.fori_loop(0, (nch + R - 1) // R, _iter, 0)
    for b in range(R):
        _scatter_wait(b)
    plsc.subcore_barrier()

    off = pl.multiple_of(si * rows_per_tile, 8)
    pltpu.sync_copy(u_sh.at[pl.ds(off, rows_per_tile), :],
                    u_out.at[ci, pl.ds(off, rows_per_tile), :])


def kernel(h, p, edge_index, W_fc, W_attn):
    n, in_dim = h.shape
    e_total = p.shape[0]
    out_dim = W_fc.shape[0]
    rel_dim = p.shape[1]

    f32 = jnp.float32
    wp = W_attn[0, out_dim:out_dim + rel_dim].reshape(rel_dim, 1)
    wsd = jnp.zeros((out_dim, 8), f32)
    wsd = wsd.at[:, 0].set(W_attn[0, :out_dim])
    wsd = wsd.at[:, 1].set(W_attn[0, out_dim + rel_dim:])
    wq = jnp.kron(jnp.eye(128 // rel_dim, dtype=f32), wp)
    p2 = p.reshape(-1, 128)
    bp8 = p2.shape[0] // (n // BH)

    grid1 = n // BH
    z, sd, q2, smax, dmax, qmax = pl.pallas_call(
        _pre_body,
        grid=(grid1,),
        in_specs=[
            pl.BlockSpec((BH, in_dim), lambda i: (i, 0)),
            pl.BlockSpec((bp8, 128), lambda i: (i, 0)),
            pl.BlockSpec((out_dim, in_dim), lambda i: (0, 0)),
            pl.BlockSpec((out_dim, 8), lambda i: (0, 0)),
            pl.BlockSpec((128, 8), lambda i: (0, 0)),
        ],
        out_specs=[
            pl.BlockSpec((BH, DP), lambda i: (i, 0)),
            pl.BlockSpec((BH, 8), lambda i: (i, 0)),
            pl.BlockSpec((bp8, 8), lambda i: (i, 0)),
            pl.BlockSpec(memory_space=pltpu.SMEM, block_shape=(1, 1),
                         index_map=lambda i: (0, 0)),
            pl.BlockSpec(memory_space=pltpu.SMEM, block_shape=(1, 1),
                         index_map=lambda i: (0, 0)),
            pl.BlockSpec(memory_space=pltpu.SMEM, block_shape=(1, 1),
                         index_map=lambda i: (0, 0)),
        ],
        out_shape=[
            jax.ShapeDtypeStruct((n, DP), f32),
            jax.ShapeDtypeStruct((n, 8), f32),
            jax.ShapeDtypeStruct((p2.shape[0], 8), f32),
            jax.ShapeDtypeStruct((1, 1), f32),
            jax.ShapeDtypeStruct((1, 1), f32),
            jax.ShapeDtypeStruct((1, 1), f32),
        ],
    )(h, p2, W_fc, wsd, wq)

    m = smax[0, 0] + dmax[0, 0] + qmax[0, 0]
    c = jnp.where(m >= 0.0, m, m * 0.01)
    cvec = jnp.full((16,), c, f32)

    d = sd[:, 1]
    q = q2.reshape(-1)
    ei = edge_index.astype(jnp.int32)
    ept = e_total // (NC * NS)

    mesh = plsc.VectorSubcoreMesh(core_axis_name="c", subcore_axis_name="s",
                                  num_cores=NC, num_subcores=NS)
    sc_params = pltpu.CompilerParams(needs_layout_passes=False,
                                     use_tc_tiling_on_sc=False)

    qd = pl.kernel(
        _qd_body,
        out_type=jax.ShapeDtypeStruct((e_total,), f32),
        mesh=mesh,
        compiler_params=sc_params,
        scratch_types=[
            pltpu.VMEM((n,), f32),
            pltpu.VMEM((ept,), jnp.int32),
            pltpu.VMEM((ept,), f32),
        ],
    )(d, ei, q)

    u = pl.kernel(
        _sc_body,
        out_type=jax.ShapeDtypeStruct((NC, NPAD, DP), f32),
        mesh=mesh,
        compiler_params=sc_params,
        scratch_types=[
            pltpu.VMEM((16,), f32),
            tuple(pltpu.VMEM((C,), jnp.int32) for _ in range(R)),
            tuple(pltpu.VMEM((C,), jnp.int32) for _ in range(R)),
            tuple(pltpu.VMEM((C,), jnp.int32) for _ in range(R)),
            tuple(pltpu.VMEM((C,), f32) for _ in range(R)),
            tuple(pltpu.VMEM((C,), f32) for _ in range(R)),
            tuple(pltpu.VMEM((C, DP), f32) for _ in range(R)),
            pltpu.VMEM_SHARED((NPAD, DP), f32),
            tuple(pltpu.SemaphoreType.DMA for _ in range(R)),
            tuple(pltpu.SemaphoreType.DMA for _ in range(R)),
            tuple(pltpu.SemaphoreType.DMA for _ in range(R)),
        ],
    )(z, qd, ei, cvec)

    h_out = pl.pallas_call(
        _post_body,
        grid=(NPAD // BN,),
        in_specs=[pl.BlockSpec((NC, BN, DP), lambda i: (0, i, 0))],
        out_specs=pl.BlockSpec((BN, out_dim), lambda i: (i, 0)),
        out_shape=jax.ShapeDtypeStruct((NPAD, out_dim), f32),
    )(u)
    return h_out[:n]


# speedup vs baseline: 1.4471x; 1.4471x over previous
import functools

import jax
import jax.numpy as jnp
from jax import lax
from jax.experimental import pallas as pl
from jax.experimental.pallas import tpu as pltpu
from jax.experimental.pallas import tpu_sc as plsc

DP = 144
NPAD = 10240
C = 80
R = 3
NC = 2
NS = 16
BH = 400
BN = 1024

E_TOTAL = 320000


def _pre_body(h_ref, pt_ref, wfc_ref, wsd_ref, wp_ref,
              z_ref, sd_ref, q_ref, smax_ref, dmax_ref, qmax_ref):
    i = pl.program_id(0)
    z = lax.dot_general(h_ref[...], wfc_ref[...], (((1,), (1,)), ((), ())),
                        preferred_element_type=jnp.float32)
    sd = jnp.dot(z, wsd_ref[...], preferred_element_type=jnp.float32)
    q = jnp.sum(pt_ref[...] * wp_ref[...], axis=0, keepdims=True)
    ones = jnp.ones((z.shape[0], 1), jnp.float32)
    zeros = jnp.zeros((z.shape[0], DP - 130), jnp.float32)
    z_ref[...] = jnp.concatenate([z, ones, sd[:, 0:1], zeros], axis=1)
    sd_ref[...] = sd
    q_ref[...] = q
    sm, dm, qm = jnp.max(sd[:, 0]), jnp.max(sd[:, 1]), jnp.max(q)

    @pl.when(i == 0)
    def _():
        smax_ref[0, 0] = sm
        dmax_ref[0, 0] = dm
        qmax_ref[0, 0] = qm

    @pl.when(i > 0)
    def _():
        smax_ref[0, 0] = jnp.maximum(smax_ref[0, 0], sm)
        dmax_ref[0, 0] = jnp.maximum(dmax_ref[0, 0], dm)
        qmax_ref[0, 0] = jnp.maximum(qmax_ref[0, 0], qm)


def _post_body(u_ref, o_ref):
    a = u_ref[0] + u_ref[1]
    num = a[:, :128]
    den = a[:, 128:129]
    o_ref[...] = jnp.where(den > 0.0, num / den, 0.0)


def _qd_body(d_hbm, ei_hbm, q_hbm, qd_out, d_v, dst_t, qd_t):
    ci = lax.axis_index("c")
    si = lax.axis_index("s")
    wid = ci * NS + si
    ept = E_TOTAL // (NC * NS)
    base = pl.multiple_of(wid * ept, 8)
    pltpu.sync_copy(d_hbm, d_v)
    pltpu.sync_copy(ei_hbm.at[1, pl.ds(base, ept)], dst_t)
    pltpu.sync_copy(q_hbm.at[pl.ds(base, ept)], qd_t)

    def _grp(i, _):
        sl = pl.ds(i * 16, 16)
        dv = plsc.load_gather(d_v, [dst_t[sl]])
        qd_t[sl] = qd_t[sl] + dv
        return 0

    lax.fori_loop(0, ept // 16, _grp, 0)
    pltpu.sync_copy(qd_t, qd_out.at[pl.ds(base, ept)])


def _sc_body(z_hbm, qd_hbm, ei_hbm, cvec_hbm,
             u_out,
             c_v, src_b, dst_b, dst_s, qd_b, w_b, rows_b, u_sh,
             lsem, gsem, ssem):
    ci = lax.axis_index("c")
    si = lax.axis_index("s")
    wid = ci * NS + si
    ept = E_TOTAL // (NC * NS)
    nch = ept // C
    rows_per_tile = NPAD // NS
    tbase = pl.multiple_of(wid * ept, 8)

    def _zrows(i, _):
        for j in range(DP // 16):
            rows_b[0][i, pl.ds(j * 16, 16)] = jnp.zeros((16,), jnp.float32)
        return 0
    lax.fori_loop(0, C, _zrows, 0)

    def _zacc(i, _):
        off = pl.multiple_of(si * rows_per_tile + i * C, 8)
        pltpu.sync_copy(rows_b[0], u_sh.at[pl.ds(off, C), :])
        return 0
    lax.fori_loop(0, rows_per_tile // C, _zacc, 0)

    pltpu.sync_copy(cvec_hbm, c_v)
    plsc.subcore_barrier()

    cvec = c_v[...]
    mask0 = jnp.where(lax.iota(jnp.int32, 16) == 0, 1.0, 0.0)

    def _lin_start(g, b):
        off = pl.ds(tbase + g * C, C)
        pltpu.async_copy(ei_hbm.at[0, off], src_b[b], lsem[b])
        pltpu.async_copy(ei_hbm.at[1, off], dst_b[b], lsem[b])
        pltpu.async_copy(qd_hbm.at[off], qd_b[b], lsem[b])

    def _lin_wait(g, b):
        off = pl.ds(tbase + g * C, C)
        pltpu.make_async_copy(ei_hbm.at[0, off], src_b[b], lsem[b]).wait()
        pltpu.make_async_copy(ei_hbm.at[1, off], dst_b[b], lsem[b]).wait()
        pltpu.make_async_copy(qd_hbm.at[off], qd_b[b], lsem[b]).wait()

    def _gather_start(b):
        pltpu.async_copy(z_hbm.at[src_b[b]], rows_b[b], gsem[b])

    def _gather_wait(b):
        pltpu.make_async_copy(z_hbm.at[src_b[b]], rows_b[b],
                              gsem[b]).wait()

    def _scatter_start(b):
        def _cp(i, _):
            sl = pl.ds(i * 16, 16)
            dst_s[b][sl] = dst_b[b][sl]
            return 0
        lax.fori_loop(0, C // 16, _cp, 0)
        pltpu.async_copy(rows_b[b], u_sh.at[dst_s[b]], ssem[b], add=True)

    def _scatter_wait(b):
        pltpu.make_async_copy(rows_b[b], u_sh.at[dst_s[b]],
                              ssem[b]).wait()

    def _compute_chunk(b):
        def _wgrp(g16, _):
            sl = pl.ds(g16 * 16, 16)
            eids = lax.iota(jnp.int32, 16) + g16 * 16
            col = jnp.full((16,), 129, jnp.int32)
            sv = plsc.load_gather(rows_b[b], [eids, col])
            a = sv + qd_b[b][sl]
            e = jnp.where(a >= 0.0, a, a * 0.01)
            w_b[b][sl] = jnp.exp(e - cvec)
            return 0
        lax.fori_loop(0, C // 16, _wgrp, 0)

        def _sgrp(g16, _):
            wv = w_b[b][pl.ds(g16 * 16, 16)]
            ebase = g16 * 16
            for lane in range(16):
                bw = jnp.full((16,), wv[lane], jnp.float32)
                ei = ebase + lane
                for j in range(128 // 16):
                    sl2 = pl.ds(j * 16, 16)
                    rows_b[b][ei, sl2] = rows_b[b][ei, sl2] * bw
                rows_b[b][ei, pl.ds(128, 16)] = bw * mask0
            return 0
        lax.fori_loop(0, C // 16, _sgrp, 0)

    for b in range(R):
        _lin_start(b, b)
    _lin_wait(0, 0)
    _gather_start(0)
    _lin_wait(1, 1)
    _gather_start(1)

    def _iter(i, _):
        for u in range(R):
            g = i * R + u
            b = u

            @pl.when(g < nch)
            def _():
                _gather_wait(b)
                _compute_chunk(b)
                _scatter_start(b)

                @pl.when(g + R < nch)
                def _():
                    _lin_start(g + R, b)

                nb = (u + 2) % R

                @pl.when((g + 2 < nch) & (g >= 1))
                def _():
                    _scatter_wait(nb)
                    _lin_wait(g + 2, nb)
                    _gather_start(nb)

                @pl.when((g + 2 < nch) & (g < 1))
                def _():
                    _lin_wait(g + 2, nb)
                    _gather_start(nb)
        return 0

    lax.fori_loop(0, (nch + R - 1) // R, _iter, 0)
    for b in range(R):
        _scatter_wait(b)
    plsc.subcore_barrier()

    off = pl.multiple_of(si * rows_per_tile, 8)
    pltpu.sync_copy(u_sh.at[pl.ds(off, rows_per_tile), :],
                    u_out.at[ci, pl.ds(off, rows_per_tile), :])


def kernel(h, p, edge_index, W_fc, W_attn):
    n, in_dim = h.shape
    e_total = p.shape[0]
    out_dim = W_fc.shape[0]
    rel_dim = p.shape[1]

    f32 = jnp.float32
    wp = W_attn[0, out_dim:out_dim + rel_dim].reshape(rel_dim, 1)
    wsd = jnp.zeros((out_dim, 8), f32)
    wsd = wsd.at[:, 0].set(W_attn[0, :out_dim])
    wsd = wsd.at[:, 1].set(W_attn[0, out_dim + rel_dim:])
    pt = p.T
    grid1 = n // BH
    bpe = e_total // grid1

    z, sd, q, smax, dmax, qmax = pl.pallas_call(
        _pre_body,
        grid=(grid1,),
        in_specs=[
            pl.BlockSpec((BH, in_dim), lambda i: (i, 0)),
            pl.BlockSpec((rel_dim, bpe), lambda i: (0, i)),
            pl.BlockSpec((out_dim, in_dim), lambda i: (0, 0)),
            pl.BlockSpec((out_dim, 8), lambda i: (0, 0)),
            pl.BlockSpec((rel_dim, 1), lambda i: (0, 0)),
        ],
        out_specs=[
            pl.BlockSpec((BH, DP), lambda i: (i, 0)),
            pl.BlockSpec((BH, 8), lambda i: (i, 0)),
            pl.BlockSpec((1, bpe), lambda i: (0, i)),
            pl.BlockSpec(memory_space=pltpu.SMEM, block_shape=(1, 1),
                         index_map=lambda i: (0, 0)),
            pl.BlockSpec(memory_space=pltpu.SMEM, block_shape=(1, 1),
                         index_map=lambda i: (0, 0)),
            pl.BlockSpec(memory_space=pltpu.SMEM, block_shape=(1, 1),
                         index_map=lambda i: (0, 0)),
        ],
        out_shape=[
            jax.ShapeDtypeStruct((n, DP), f32),
            jax.ShapeDtypeStruct((n, 8), f32),
            jax.ShapeDtypeStruct((1, e_total), f32),
            jax.ShapeDtypeStruct((1, 1), f32),
            jax.ShapeDtypeStruct((1, 1), f32),
            jax.ShapeDtypeStruct((1, 1), f32),
        ],
    )(h, pt, W_fc, wsd, wp)

    m = smax[0, 0] + dmax[0, 0] + qmax[0, 0]
    c = jnp.where(m >= 0.0, m, m * 0.01)
    cvec = jnp.full((16,), c, f32)

    d = sd[:, 1]
    q = q.reshape(-1)
    ei = edge_index.astype(jnp.int32)
    ept = e_total // (NC * NS)

    mesh = plsc.VectorSubcoreMesh(core_axis_name="c", subcore_axis_name="s",
                                  num_cores=NC, num_subcores=NS)
    sc_params = pltpu.CompilerParams(needs_layout_passes=False,
                                     use_tc_tiling_on_sc=False)

    qd = pl.kernel(
        _qd_body,
        out_type=jax.ShapeDtypeStruct((e_total,), f32),
        mesh=mesh,
        compiler_params=sc_params,
        scratch_types=[
            pltpu.VMEM((n,), f32),
            pltpu.VMEM((ept,), jnp.int32),
            pltpu.VMEM((ept,), f32),
        ],
    )(d, ei, q)

    u = pl.kernel(
        _sc_body,
        out_type=jax.ShapeDtypeStruct((NC, NPAD, DP), f32),
        mesh=mesh,
        compiler_params=sc_params,
        scratch_types=[
            pltpu.VMEM((16,), f32),
            tuple(pltpu.VMEM((C,), jnp.int32) for _ in range(R)),
            tuple(pltpu.VMEM((C,), jnp.int32) for _ in range(R)),
            tuple(pltpu.VMEM((C,), jnp.int32) for _ in range(R)),
            tuple(pltpu.VMEM((C,), f32) for _ in range(R)),
            tuple(pltpu.VMEM((C,), f32) for _ in range(R)),
            tuple(pltpu.VMEM((C, DP), f32) for _ in range(R)),
            pltpu.VMEM_SHARED((NPAD, DP), f32),
            tuple(pltpu.SemaphoreType.DMA for _ in range(R)),
            tuple(pltpu.SemaphoreType.DMA for _ in range(R)),
            tuple(pltpu.SemaphoreType.DMA for _ in range(R)),
        ],
    )(z, qd, ei, cvec)

    h_out = pl.pallas_call(
        _post_body,
        grid=(NPAD // BN,),
        in_specs=[pl.BlockSpec((NC, BN, DP), lambda i: (0, i, 0))],
        out_specs=pl.BlockSpec((BN, out_dim), lambda i: (i, 0)),
        out_shape=jax.ShapeDtypeStruct((NPAD, out_dim), f32),
    )(u)
    return h_out[:n]


# speedup vs baseline: 1.4486x; 1.0011x over previous
import functools

import jax
import jax.numpy as jnp
from jax import lax
from jax.experimental import pallas as pl
from jax.experimental.pallas import tpu as pltpu
from jax.experimental.pallas import tpu_sc as plsc

DP = 144
NPAD = 10240
C = 80
R = 3
NC = 2
NS = 16
BH = 400
BN = 1024

E_TOTAL = 320000


def _pre_body(h_ref, pt_ref, wfc_ref, wsd_ref, wp_ref,
              z_ref, sd_ref, q_ref, smax_ref, dmax_ref, qmax_ref):
    i = pl.program_id(0)
    z = lax.dot_general(h_ref[...], wfc_ref[...], (((1,), (1,)), ((), ())),
                        preferred_element_type=jnp.float32)
    sd = jnp.dot(z, wsd_ref[...], preferred_element_type=jnp.float32)
    q = jnp.sum(pt_ref[...] * wp_ref[...], axis=0, keepdims=True)
    ones = jnp.ones((z.shape[0], 1), jnp.float32)
    zeros = jnp.zeros((z.shape[0], DP - 130), jnp.float32)
    z_ref[...] = jnp.concatenate([z, ones, sd[:, 0:1], zeros], axis=1)
    sd_ref[...] = sd
    q_ref[...] = q
    sm, dm, qm = jnp.max(sd[:, 0]), jnp.max(sd[:, 1]), jnp.max(q)

    @pl.when(i == 0)
    def _():
        smax_ref[0, 0] = sm
        dmax_ref[0, 0] = dm
        qmax_ref[0, 0] = qm

    @pl.when(i > 0)
    def _():
        smax_ref[0, 0] = jnp.maximum(smax_ref[0, 0], sm)
        dmax_ref[0, 0] = jnp.maximum(dmax_ref[0, 0], dm)
        qmax_ref[0, 0] = jnp.maximum(qmax_ref[0, 0], qm)


def _post_body(u_ref, o_ref):
    a = u_ref[0] + u_ref[1]
    num = a[:, :128]
    den = a[:, 128:129]
    o_ref[...] = jnp.where(den > 0.0, num / den, 0.0)


def _qd_body(d_hbm, ei_hbm, q_hbm, qd_out, d_v, dst_t, qd_t):
    ci = lax.axis_index("c")
    si = lax.axis_index("s")
    wid = ci * NS + si
    ept = E_TOTAL // (NC * NS)
    base = pl.multiple_of(wid * ept, 8)
    pltpu.sync_copy(d_hbm, d_v)
    pltpu.sync_copy(ei_hbm.at[1, pl.ds(base, ept)], dst_t)
    pltpu.sync_copy(q_hbm.at[0, pl.ds(base, ept)], qd_t)

    def _grp(i, _):
        sl = pl.ds(i * 16, 16)
        dv = plsc.load_gather(d_v, [dst_t[sl]])
        qd_t[sl] = qd_t[sl] + dv
        return 0

    lax.fori_loop(0, ept // 16, _grp, 0)
    pltpu.sync_copy(qd_t, qd_out.at[pl.ds(base, ept)])


def _sc_body(z_hbm, qd_hbm, ei_hbm, cvec_hbm,
             u_out,
             c_v, src_b, dst_b, dst_s, qd_b, w_b, rows_b, u_sh,
             lsem, gsem, ssem):
    ci = lax.axis_index("c")
    si = lax.axis_index("s")
    wid = ci * NS + si
    ept = E_TOTAL // (NC * NS)
    nch = ept // C
    rows_per_tile = NPAD // NS
    tbase = pl.multiple_of(wid * ept, 8)

    def _zrows(i, _):
        for j in range(DP // 16):
            rows_b[0][i, pl.ds(j * 16, 16)] = jnp.zeros((16,), jnp.float32)
        return 0
    lax.fori_loop(0, C, _zrows, 0)

    def _zacc(i, _):
        off = pl.multiple_of(si * rows_per_tile + i * C, 8)
        pltpu.sync_copy(rows_b[0], u_sh.at[pl.ds(off, C), :])
        return 0
    lax.fori_loop(0, rows_per_tile // C, _zacc, 0)

    pltpu.sync_copy(cvec_hbm, c_v)
    plsc.subcore_barrier()

    cvec = c_v[...]
    mask0 = jnp.where(lax.iota(jnp.int32, 16) == 0, 1.0, 0.0)

    def _lin_start(g, b):
        off = pl.ds(tbase + g * C, C)
        pltpu.async_copy(ei_hbm.at[0, off], src_b[b], lsem[b])
        pltpu.async_copy(ei_hbm.at[1, off], dst_b[b], lsem[b])
        pltpu.async_copy(qd_hbm.at[off], qd_b[b], lsem[b])

    def _lin_wait(g, b):
        off = pl.ds(tbase + g * C, C)
        pltpu.make_async_copy(ei_hbm.at[0, off], src_b[b], lsem[b]).wait()
        pltpu.make_async_copy(ei_hbm.at[1, off], dst_b[b], lsem[b]).wait()
        pltpu.make_async_copy(qd_hbm.at[off], qd_b[b], lsem[b]).wait()

    def _gather_start(b):
        pltpu.async_copy(z_hbm.at[src_b[b]], rows_b[b], gsem[b])

    def _gather_wait(b):
        pltpu.make_async_copy(z_hbm.at[src_b[b]], rows_b[b],
                              gsem[b]).wait()

    def _scatter_start(b):
        def _cp(i, _):
            sl = pl.ds(i * 16, 16)
            dst_s[b][sl] = dst_b[b][sl]
            return 0
        lax.fori_loop(0, C // 16, _cp, 0)
        pltpu.async_copy(rows_b[b], u_sh.at[dst_s[b]], ssem[b], add=True)

    def _scatter_wait(b):
        pltpu.make_async_copy(rows_b[b], u_sh.at[dst_s[b]],
                              ssem[b]).wait()

    def _compute_chunk(b):
        def _wgrp(g16, _):
            sl = pl.ds(g16 * 16, 16)
            eids = lax.iota(jnp.int32, 16) + g16 * 16
            col = jnp.full((16,), 129, jnp.int32)
            sv = plsc.load_gather(rows_b[b], [eids, col])
            a = sv + qd_b[b][sl]
            e = jnp.where(a >= 0.0, a, a * 0.01)
            w_b[b][sl] = jnp.exp(e - cvec)
            return 0
        lax.fori_loop(0, C // 16, _wgrp, 0)

        def _sgrp(g16, _):
            wv = w_b[b][pl.ds(g16 * 16, 16)]
            ebase = g16 * 16
            for lane in range(16):
                bw = jnp.full((16,), wv[lane], jnp.float32)
                ei = ebase + lane
                for j in range(128 // 16):
                    sl2 = pl.ds(j * 16, 16)
                    rows_b[b][ei, sl2] = rows_b[b][ei, sl2] * bw
                rows_b[b][ei, pl.ds(128, 16)] = bw * mask0
            return 0
        lax.fori_loop(0, C // 16, _sgrp, 0)

    for b in range(R):
        _lin_start(b, b)
    _lin_wait(0, 0)
    _gather_start(0)
    _lin_wait(1, 1)
    _gather_start(1)

    def _iter(i, _):
        for u in range(R):
            g = i * R + u
            b = u

            @pl.when(g < nch)
            def _():
                _gather_wait(b)
                _compute_chunk(b)
                _scatter_start(b)

                @pl.when(g + R < nch)
                def _():
                    _lin_start(g + R, b)

                nb = (u + 2) % R

                @pl.when((g + 2 < nch) & (g >= 1))
                def _():
                    _scatter_wait(nb)
                    _lin_wait(g + 2, nb)
                    _gather_start(nb)

                @pl.when((g + 2 < nch) & (g < 1))
                def _():
                    _lin_wait(g + 2, nb)
                    _gather_start(nb)
        return 0

    lax.fori_loop(0, (nch + R - 1) // R, _iter, 0)
    for b in range(R):
        _scatter_wait(b)
    plsc.subcore_barrier()

    off = pl.multiple_of(si * rows_per_tile, 8)
    pltpu.sync_copy(u_sh.at[pl.ds(off, rows_per_tile), :],
                    u_out.at[ci, pl.ds(off, rows_per_tile), :])


def kernel(h, p, edge_index, W_fc, W_attn):
    n, in_dim = h.shape
    e_total = p.shape[0]
    out_dim = W_fc.shape[0]
    rel_dim = p.shape[1]

    f32 = jnp.float32
    wp = W_attn[0, out_dim:out_dim + rel_dim].reshape(rel_dim, 1)
    wsd = jnp.zeros((out_dim, 8), f32)
    wsd = wsd.at[:, 0].set(W_attn[0, :out_dim])
    wsd = wsd.at[:, 1].set(W_attn[0, out_dim + rel_dim:])
    pt = p.T
    grid1 = n // BH
    bpe = e_total // grid1

    z, sd, q, smax, dmax, qmax = pl.pallas_call(
        _pre_body,
        grid=(grid1,),
        in_specs=[
            pl.BlockSpec((BH, in_dim), lambda i: (i, 0)),
            pl.BlockSpec((rel_dim, bpe), lambda i: (0, i)),
            pl.BlockSpec((out_dim, in_dim), lambda i: (0, 0)),
            pl.BlockSpec((out_dim, 8), lambda i: (0, 0)),
            pl.BlockSpec((rel_dim, 1), lambda i: (0, 0)),
        ],
        out_specs=[
            pl.BlockSpec((BH, DP), lambda i: (i, 0)),
            pl.BlockSpec((BH, 8), lambda i: (i, 0)),
            pl.BlockSpec((1, bpe), lambda i: (0, i)),
            pl.BlockSpec(memory_space=pltpu.SMEM, block_shape=(1, 1),
                         index_map=lambda i: (0, 0)),
            pl.BlockSpec(memory_space=pltpu.SMEM, block_shape=(1, 1),
                         index_map=lambda i: (0, 0)),
            pl.BlockSpec(memory_space=pltpu.SMEM, block_shape=(1, 1),
                         index_map=lambda i: (0, 0)),
        ],
        out_shape=[
            jax.ShapeDtypeStruct((n, DP), f32),
            jax.ShapeDtypeStruct((n, 8), f32),
            jax.ShapeDtypeStruct((1, e_total), f32),
            jax.ShapeDtypeStruct((1, 1), f32),
            jax.ShapeDtypeStruct((1, 1), f32),
            jax.ShapeDtypeStruct((1, 1), f32),
        ],
    )(h, pt, W_fc, wsd, wp)

    m = smax[0, 0] + dmax[0, 0] + qmax[0, 0]
    c = jnp.where(m >= 0.0, m, m * 0.01)
    cvec = jnp.full((16,), c, f32)

    d = sd[:, 1]
    ei = edge_index.astype(jnp.int32)
    ept = e_total // (NC * NS)

    mesh = plsc.VectorSubcoreMesh(core_axis_name="c", subcore_axis_name="s",
                                  num_cores=NC, num_subcores=NS)
    sc_params = pltpu.CompilerParams(needs_layout_passes=False,
                                     use_tc_tiling_on_sc=False)

    qd = pl.kernel(
        _qd_body,
        out_type=jax.ShapeDtypeStruct((e_total,), f32),
        mesh=mesh,
        compiler_params=sc_params,
        scratch_types=[
            pltpu.VMEM((n,), f32),
            pltpu.VMEM((ept,), jnp.int32),
            pltpu.VMEM((ept,), f32),
        ],
    )(d, ei, q)

    u = pl.kernel(
        _sc_body,
        out_type=jax.ShapeDtypeStruct((NC, NPAD, DP), f32),
        mesh=mesh,
        compiler_params=sc_params,
        scratch_types=[
            pltpu.VMEM((16,), f32),
            tuple(pltpu.VMEM((C,), jnp.int32) for _ in range(R)),
            tuple(pltpu.VMEM((C,), jnp.int32) for _ in range(R)),
            tuple(pltpu.VMEM((C,), jnp.int32) for _ in range(R)),
            tuple(pltpu.VMEM((C,), f32) for _ in range(R)),
            tuple(pltpu.VMEM((C,), f32) for _ in range(R)),
            tuple(pltpu.VMEM((C, DP), f32) for _ in range(R)),
            pltpu.VMEM_SHARED((NPAD, DP), f32),
            tuple(pltpu.SemaphoreType.DMA for _ in range(R)),
            tuple(pltpu.SemaphoreType.DMA for _ in range(R)),
            tuple(pltpu.SemaphoreType.DMA for _ in range(R)),
        ],
    )(z, qd, ei, cvec)

    h_out = pl.pallas_call(
        _post_body,
        grid=(NPAD // BN,),
        in_specs=[pl.BlockSpec((NC, BN, DP), lambda i: (0, i, 0))],
        out_specs=pl.BlockSpec((BN, out_dim), lambda i: (i, 0)),
        out_shape=jax.ShapeDtypeStruct((NPAD, out_dim), f32),
    )(u)
    return h_out[:n]
